# baseline (speedup 1.0000x reference)
import jax
import jax.numpy as jnp
from jax.experimental import pallas as pl
from jax.experimental.pallas import tpu as pltpu

TOTAL = 32768
D = 256
N_OUT = TOTAL - 2
RS_OUT = TOTAL - 1
BLK = 16296


def _copy_body(x_ref, rs_ref, data_ref, rs_out_ref):
    data_ref[...] = x_ref[...]
    i = pl.program_id(0)

    @pl.when(i == 0)
    def _():
        rs_out_ref[...] = rs_ref[pl.ds(0, RS_OUT)]


def kernel(x_data, x_row_splits):
    grid = (pl.cdiv(N_OUT, BLK),)
    data, rs = pl.pallas_call(
        _copy_body,
        grid=grid,
        in_specs=[
            pl.BlockSpec((BLK, D), lambda i: (i, 0)),
            pl.BlockSpec((TOTAL,), lambda i: (0,)),
        ],
        out_specs=[
            pl.BlockSpec((BLK, D), lambda i: (i, 0)),
            pl.BlockSpec((RS_OUT,), lambda i: (0,)),
        ],
        out_shape=[
            jax.ShapeDtypeStruct((N_OUT, D), jnp.float32),
            jax.ShapeDtypeStruct((RS_OUT,), jnp.int32),
        ],
        compiler_params=pltpu.CompilerParams(
            vmem_limit_bytes=100 * 1024 * 1024),
    )(x_data, x_row_splits)
    return (data, rs)


# speedup vs baseline: 1.0408x; 1.0408x over previous
import jax
import jax.numpy as jnp
from jax.experimental import pallas as pl
from jax.experimental.pallas import tpu as pltpu

TOTAL = 32768
D = 256
N_OUT = TOTAL - 2
RS_OUT = TOTAL - 1
BLK = 15600


def _copy_body(x_ref, rs_ref, data_ref, rs_out_ref):
    data_ref[...] = x_ref[...]
    i = pl.program_id(0)

    @pl.when(i == 0)
    def _():
        rs_out_ref[...] = rs_ref[pl.ds(0, RS_OUT)]


def kernel(x_data, x_row_splits):
    grid = (pl.cdiv(N_OUT, BLK),)
    data, rs = pl.pallas_call(
        _copy_body,
        grid=grid,
        in_specs=[
            pl.BlockSpec((BLK, D), lambda i: (i, 0)),
            pl.BlockSpec((TOTAL,), lambda i: (0,)),
        ],
        out_specs=[
            pl.BlockSpec((BLK, D), lambda i: (i, 0)),
            pl.BlockSpec((RS_OUT,), lambda i: (0,)),
        ],
        out_shape=[
            jax.ShapeDtypeStruct((N_OUT, D), jnp.float32),
            jax.ShapeDtypeStruct((RS_OUT,), jnp.int32),
        ],
        compiler_params=pltpu.CompilerParams(
            vmem_limit_bytes=100 * 1024 * 1024),
    )(x_data, x_row_splits)
    return (data, rs)


# speedup vs baseline: 1.0409x; 1.0001x over previous
import jax
import jax.numpy as jnp
from jax.experimental import pallas as pl
from jax.experimental.pallas import tpu as pltpu

TOTAL = 32768
D = 256
N_OUT = TOTAL - 2
RS_OUT = TOTAL - 1
BLK = 14000


def _copy_body(x_ref, rs_ref, data_ref, rs_out_ref):
    data_ref[...] = x_ref[...]
    i = pl.program_id(0)

    @pl.when(i == 0)
    def _():
        rs_out_ref[...] = rs_ref[pl.ds(0, RS_OUT)]


def kernel(x_data, x_row_splits):
    grid = (pl.cdiv(N_OUT, BLK),)
    data, rs = pl.pallas_call(
        _copy_body,
        grid=grid,
        in_specs=[
            pl.BlockSpec((BLK, D), lambda i: (i, 0)),
            pl.BlockSpec((TOTAL,), lambda i: (0,)),
        ],
        out_specs=[
            pl.BlockSpec((BLK, D), lambda i: (i, 0)),
            pl.BlockSpec((RS_OUT,), lambda i: (0,)),
        ],
        out_shape=[
            jax.ShapeDtypeStruct((N_OUT, D), jnp.float32),
            jax.ShapeDtypeStruct((RS_OUT,), jnp.int32),
        ],
        compiler_params=pltpu.CompilerParams(
            vmem_limit_bytes=100 * 1024 * 1024),
    )(x_data, x_row_splits)
    return (data, rs)


# speedup vs baseline: 1.0494x; 1.0082x over previous
import jax
import jax.numpy as jnp
from jax.experimental import pallas as pl

TOTAL = 32768
D = 256
N_OUT = TOTAL - 2
RS_OUT = TOTAL - 1
BLK = 14936


def _copy_body(x_ref, rs_ref, data_ref, rs_out_ref):
    data_ref[...] = x_ref[...]
    i = pl.program_id(0)

    @pl.when(i == 0)
    def _():
        rs_out_ref[...] = rs_ref[pl.ds(0, RS_OUT)]


def kernel(x_data, x_row_splits):
    grid = (pl.cdiv(N_OUT, BLK),)
    data, rs = pl.pallas_call(
        _copy_body,
        grid=grid,
        in_specs=[
            pl.BlockSpec((BLK, D), lambda i: (i, 0)),
            pl.BlockSpec((TOTAL,), lambda i: (0,)),
        ],
        out_specs=[
            pl.BlockSpec((BLK, D), lambda i: (i, 0)),
            pl.BlockSpec((RS_OUT,), lambda i: (0,)),
        ],
        out_shape=[
            jax.ShapeDtypeStruct((N_OUT, D), jnp.float32),
            jax.ShapeDtypeStruct((RS_OUT,), jnp.int32),
        ],
    )(x_data, x_row_splits)
    return (data, rs)
